# baseline (speedup 1.0000x reference)
import functools

import jax
import jax.numpy as jnp
from jax import lax
from jax.experimental import pallas as pl
from jax.experimental.pallas import tpu as pltpu
from jax.experimental.pallas import tpu_sc as plsc

_NC = 2
_NS = 16
_NW = _NC * _NS
_C = 80
_CS = 80
_KD = 10


def _sc_mesh():
    return plsc.VectorSubcoreMesh(core_axis_name="c", subcore_axis_name="s",
                                  num_cores=_NC, num_subcores=_NS)


def _worker_id():
    c = lax.axis_index("c")
    s = lax.axis_index("s")
    return c * _NS + s, c, s


def _degrees(src, dst, n_pad):
    e = src.shape[0]
    epw = e // _NS
    n_chunks = epw // _C
    assert n_chunks * _C == epw and n_chunks % _KD == 0
    rows_per_sub = n_pad // _NS
    assert rows_per_sub % 8 == 0
    n_batches = n_chunks // _KD

    ones = jnp.ones((_C, 128), dtype=jnp.float32)
    zeros = jnp.zeros((rows_per_sub, 128), dtype=jnp.float32)
    sd4 = jnp.concatenate([src, dst]).reshape(_NC, _NS, n_chunks, _C)

    @functools.partial(
        pl.kernel,
        out_type=jax.ShapeDtypeStruct((_NC, n_pad, 128), jnp.float32),
        mesh=_sc_mesh(),
        scratch_types=[
            pltpu.VMEM((n_chunks, _C), jnp.int32),
            pltpu.VMEM((_C, 128), jnp.float32),
            pltpu.VMEM_SHARED((n_pad, 128), jnp.float32),
            pltpu.SemaphoreType.DMA,
        ],
    )
    def k(sd_h, ones_h, zeros_h, degs_h, idx_all, ones_v, acc_sh, ssem):
        _, c, s = _worker_id()
        r0 = s * rows_per_sub
        pltpu.sync_copy(zeros_h, acc_sh.at[pl.ds(r0, rows_per_sub)])
        pltpu.sync_copy(ones_h, ones_v)
        pltpu.sync_copy(sd_h.at[c, s], idx_all)
        plsc.subcore_barrier()

        def fire(j):
            pltpu.async_copy(ones_v, acc_sh.at[idx_all.at[j]], ssem, add=True)

        def drain():
            pltpu.make_async_copy(zeros_h.at[pl.ds(0, _C)], ones_v, ssem).wait()

        for b in range(_KD):
            fire(b)

        def body(i, carry):
            for b in range(_KD):
                fire((i + 1) * _KD + b)
            for b in range(_KD):
                drain()
            return carry

        lax.fori_loop(0, n_batches - 1, body, 0)
        for b in range(_KD):
            drain()
        plsc.subcore_barrier()
        pltpu.sync_copy(acc_sh.at[pl.ds(r0, rows_per_sub)],
                        degs_h.at[c, pl.ds(r0, rows_per_sub)])

    return k(sd4, ones, zeros)


def _spmm(x, packed, n_pad):
    n_nodes, d = x.shape
    nw, nc, cs = packed.shape
    assert nw == _NW and cs == _CS and nc % 3 == 0 and nc >= 6
    rows_per_sub = n_pad // _NS
    zeros = jnp.zeros((rows_per_sub, d), dtype=jnp.float32)

    @functools.partial(
        pl.kernel,
        out_type=jax.ShapeDtypeStruct((_NC, n_pad, d), jnp.float32),
        mesh=_sc_mesh(),
        scratch_types=[
            pltpu.VMEM((nc, _CS), jnp.int32),
            pltpu.VMEM((6, _CS), jnp.int32),
            pltpu.VMEM((_CS, d), jnp.float32),
            pltpu.VMEM((_CS, d), jnp.float32),
            pltpu.VMEM((_CS, d), jnp.float32),
            pltpu.SemaphoreType.DMA,
            pltpu.SemaphoreType.DMA,
            pltpu.SemaphoreType.DMA,
            pltpu.SemaphoreType.DMA,
            pltpu.SemaphoreType.DMA,
            pltpu.SemaphoreType.DMA,
            pltpu.VMEM_SHARED((n_pad, d), jnp.float32),
        ],
    )
    def k(x_h, packed_h, zeros_h, part_h,
          pidx, idx, r0b, r1b, r2b, g0s, g1s, g2s, s0s, s1s, s2s, acc_sh):
        w, c, s = _worker_id()
        rw0 = s * rows_per_sub
        pltpu.sync_copy(packed_h.at[w], pidx)
        pltpu.sync_copy(zeros_h, acc_sh.at[pl.ds(rw0, rows_per_sub)])
        plsc.subcore_barrier()

        rows = (r0b, r1b, r2b)
        gsem = (g0s, g1s, g2s)
        ssem = (s0s, s1s, s2s)

        def unpack(j, b):
            for t in range(_CS // 16):
                v = pidx[j, pl.ds(t * 16, 16)]
                idx[2 * b, pl.ds(t * 16, 16)] = lax.bitwise_and(v, 0xFFFF)
                idx[2 * b + 1, pl.ds(t * 16, 16)] = (
                    lax.shift_right_logical(v, 16))

        def gather(j, b):
            unpack(j, b)
            pltpu.async_copy(x_h.at[idx.at[2 * b]], rows[b], gsem[b])

        def scatter(b):
            pltpu.async_copy(rows[b], acc_sh.at[idx.at[2 * b + 1]],
                             ssem[b], add=True)

        def drain(b, sems):
            pltpu.make_async_copy(zeros_h.at[pl.ds(0, _CS)],
                                  rows[b], sems[b]).wait()

        gather(0, 0)
        gather(1, 1)
        drain(0, gsem)
        scatter(0)
        gather(2, 2)
        drain(1, gsem)
        scatter(1)
        drain(0, ssem)
        gather(3, 0)
        drain(2, gsem)
        scatter(2)
        drain(1, ssem)
        gather(4, 1)

        def body(i, carry):
            g0 = 3 * i
            drain(0, gsem)
            scatter(0)
            drain(2, ssem)
            gather(g0 + 2, 2)
            drain(1, gsem)
            scatter(1)
            drain(0, ssem)
            gather(g0 + 3, 0)
            drain(2, gsem)
            scatter(2)
            drain(1, ssem)
            gather(g0 + 4, 1)
            return carry

        lax.fori_loop(1, nc // 3 - 1, body, 0)
        drain(0, gsem)
        scatter(0)
        drain(2, ssem)
        gather(nc - 1, 2)
        drain(1, gsem)
        scatter(1)
        drain(0, ssem)
        drain(2, gsem)
        scatter(2)
        drain(1, ssem)
        drain(2, ssem)
        plsc.subcore_barrier()
        pltpu.sync_copy(acc_sh.at[pl.ds(rw0, rows_per_sub)],
                        part_h.at[c, pl.ds(rw0, rows_per_sub)])

    return k(x, packed, zeros)


def _tc_lin(h, L, b):
    n = h.shape[0]
    d_out = L.shape[1]
    has_b = b is not None

    def body(h_r, l_r, *rest):
        if has_b:
            b_r, y_r = rest
        else:
            (y_r,) = rest
        y = jnp.dot(h_r[...], l_r[...], preferred_element_type=jnp.float32)
        if has_b:
            y = y + b_r[...]
        y_r[...] = y

    args = [h, L] + ([b.reshape(1, -1)] if has_b else [])
    return pl.pallas_call(
        body,
        out_shape=jax.ShapeDtypeStruct((n, d_out), jnp.float32),
    )(*args)


def _tc_prep(feat, degs):
    n, d = feat.shape

    def body(feat_r, degs_r, x0_r, ns_r, nd_r):
        deg_o = degs_r[0, :n, 0:1]
        deg_i = degs_r[1, :n, 0:1]
        ns = lax.rsqrt(jnp.maximum(deg_o, 1.0))
        nd = lax.rsqrt(jnp.maximum(deg_i, 1.0))
        x0_r[...] = feat_r[...] * ns
        ns_r[...] = ns
        nd_r[...] = nd

    return pl.pallas_call(
        body,
        out_shape=[
            jax.ShapeDtypeStruct((n, d), jnp.float32),
            jax.ShapeDtypeStruct((n, 1), jnp.float32),
            jax.ShapeDtypeStruct((n, 1), jnp.float32),
        ],
    )(feat, degs)


def _tc_mid(part, nd, ns, W, y_prev):
    n, d = y_prev.shape

    def body(part_r, nd_r, ns_r, w_r, y_r, h_r, xn_r):
        agg = (part_r[0, :n] + part_r[1, :n]) * nd_r[...]
        h = jnp.dot(agg, w_r[...], preferred_element_type=jnp.float32) + y_r[...]
        h_r[...] = h
        xn_r[...] = h * ns_r[...]

    return pl.pallas_call(
        body,
        out_shape=[
            jax.ShapeDtypeStruct((n, d), jnp.float32),
            jax.ShapeDtypeStruct((n, d), jnp.float32),
        ],
    )(part, nd, ns, W, y_prev)


def _tc_final(part, nd, W, y_prev):
    n, d_out = y_prev.shape

    def body(part_r, nd_r, w_r, y_r, out_r):
        agg = (part_r[0, :n] + part_r[1, :n]) * nd_r[...]
        out_r[...] = (jnp.dot(agg, w_r[...], preferred_element_type=jnp.float32)
                      + y_r[...])

    return pl.pallas_call(
        body,
        out_shape=jax.ShapeDtypeStruct((n, d_out), jnp.float32),
    )(part, nd, W, y_prev)


def kernel(feat, edge_index, W0, W1, W2, b2, L0, L1, L2):
    n, d = feat.shape
    n_pad = ((n + 127) // 128) * 128
    src = edge_index[0]
    dst = edge_index[1]
    e = src.shape[0]
    chunk_sz = _NW * _CS
    nc = -(-e // chunk_sz)
    nc += (-nc) % 3
    e_pad = nc * chunk_sz
    pad_i = jnp.arange(e_pad - e, dtype=jnp.int32)
    src_p = jnp.concatenate([src, pad_i % n])
    dst_p = jnp.concatenate([dst, n + pad_i % (n_pad - n)])
    packed = (src_p | (dst_p << 16)).reshape(_NW, nc, _CS)

    y0 = _tc_lin(feat, L0, None)
    degs = _degrees(src, dst, n_pad)
    x0, ns, nd = _tc_prep(feat, degs)

    p0 = _spmm(x0, packed, n_pad)
    h1, x1 = _tc_mid(p0, nd, ns, W0, y0)

    y1 = _tc_lin(h1, L1, None)
    p1 = _spmm(x1, packed, n_pad)
    h2, x2 = _tc_mid(p1, nd, ns, W1, y1)

    y2 = _tc_lin(h2, L2, b2)
    p2 = _spmm(x2, packed, n_pad)
    return _tc_final(p2, nd, W2, y2)


# speedup vs baseline: 1.0133x; 1.0133x over previous
import functools

import jax
import jax.numpy as jnp
from jax import lax
from jax.experimental import pallas as pl
from jax.experimental.pallas import tpu as pltpu
from jax.experimental.pallas import tpu_sc as plsc

_NC = 2
_NS = 16
_NW = _NC * _NS
_C = 80
_CS = 80
_KD = 25


def _sc_mesh():
    return plsc.VectorSubcoreMesh(core_axis_name="c", subcore_axis_name="s",
                                  num_cores=_NC, num_subcores=_NS)


def _worker_id():
    c = lax.axis_index("c")
    s = lax.axis_index("s")
    return c * _NS + s, c, s


def _degrees(src, dst, n_pad):
    e = src.shape[0]
    epw = e // _NS
    n_chunks = epw // _C
    assert n_chunks * _C == epw and n_chunks % _KD == 0
    rows_per_sub = n_pad // _NS
    assert rows_per_sub % 8 == 0
    n_batches = n_chunks // _KD

    ones = jnp.ones((_C, 128), dtype=jnp.float32)
    zeros = jnp.zeros((rows_per_sub, 128), dtype=jnp.float32)
    sd4 = jnp.concatenate([src, dst]).reshape(_NC, _NS, n_chunks, _C)

    @functools.partial(
        pl.kernel,
        out_type=jax.ShapeDtypeStruct((_NC, n_pad, 128), jnp.float32),
        mesh=_sc_mesh(),
        scratch_types=[
            pltpu.VMEM((n_chunks, _C), jnp.int32),
            pltpu.VMEM((_C, 128), jnp.float32),
            pltpu.VMEM_SHARED((n_pad, 128), jnp.float32),
            pltpu.SemaphoreType.DMA,
        ],
    )
    def k(sd_h, ones_h, zeros_h, degs_h, idx_all, ones_v, acc_sh, ssem):
        _, c, s = _worker_id()
        r0 = s * rows_per_sub
        pltpu.sync_copy(zeros_h, acc_sh.at[pl.ds(r0, rows_per_sub)])
        pltpu.sync_copy(ones_h, ones_v)
        pltpu.sync_copy(sd_h.at[c, s], idx_all)
        plsc.subcore_barrier()

        def fire(j):
            pltpu.async_copy(ones_v, acc_sh.at[idx_all.at[j]], ssem, add=True)

        def drain():
            pltpu.make_async_copy(zeros_h.at[pl.ds(0, _C)], ones_v, ssem).wait()

        for b in range(_KD):
            fire(b)

        def body(i, carry):
            for b in range(_KD):
                fire((i + 1) * _KD + b)
            for b in range(_KD):
                drain()
            return carry

        lax.fori_loop(0, n_batches - 1, body, 0)
        for b in range(_KD):
            drain()
        plsc.subcore_barrier()
        pltpu.sync_copy(acc_sh.at[pl.ds(r0, rows_per_sub)],
                        degs_h.at[c, pl.ds(r0, rows_per_sub)])

    return k(sd4, ones, zeros)


def _spmm(x, packed, n_pad):
    n_nodes, d = x.shape
    nw, nc, cs = packed.shape
    assert nw == _NW and cs == _CS and nc % 3 == 0 and nc >= 6
    rows_per_sub = n_pad // _NS
    zeros = jnp.zeros((rows_per_sub, d), dtype=jnp.float32)

    @functools.partial(
        pl.kernel,
        out_type=jax.ShapeDtypeStruct((_NC, n_pad, d), jnp.float32),
        mesh=_sc_mesh(),
        scratch_types=[
            pltpu.VMEM((nc, _CS), jnp.int32),
            pltpu.VMEM((6, _CS), jnp.int32),
            pltpu.VMEM((_CS, d), jnp.float32),
            pltpu.VMEM((_CS, d), jnp.float32),
            pltpu.VMEM((_CS, d), jnp.float32),
            pltpu.SemaphoreType.DMA,
            pltpu.SemaphoreType.DMA,
            pltpu.SemaphoreType.DMA,
            pltpu.SemaphoreType.DMA,
            pltpu.SemaphoreType.DMA,
            pltpu.SemaphoreType.DMA,
            pltpu.VMEM_SHARED((n_pad, d), jnp.float32),
        ],
    )
    def k(x_h, packed_h, zeros_h, part_h,
          pidx, idx, r0b, r1b, r2b, g0s, g1s, g2s, s0s, s1s, s2s, acc_sh):
        w, c, s = _worker_id()
        rw0 = s * rows_per_sub
        pltpu.sync_copy(packed_h.at[w], pidx)

        rows = (r0b, r1b, r2b)
        gsem = (g0s, g1s, g2s)
        ssem = (s0s, s1s, s2s)

        def unpack(j, b):
            for t in range(_CS // 16):
                v = pidx[j, pl.ds(t * 16, 16)]
                idx[2 * b, pl.ds(t * 16, 16)] = lax.bitwise_and(v, 0xFFFF)
                idx[2 * b + 1, pl.ds(t * 16, 16)] = (
                    lax.shift_right_logical(v, 16))

        def gather(j, b):
            unpack(j, b)
            pltpu.async_copy(x_h.at[idx.at[2 * b]], rows[b], gsem[b])

        def scatter(b):
            pltpu.async_copy(rows[b], acc_sh.at[idx.at[2 * b + 1]],
                             ssem[b], add=True)

        def drain(b, sems):
            pltpu.make_async_copy(zeros_h.at[pl.ds(0, _CS)],
                                  rows[b], sems[b]).wait()

        gather(0, 0)
        gather(1, 1)
        pltpu.sync_copy(zeros_h, acc_sh.at[pl.ds(rw0, rows_per_sub)])
        plsc.subcore_barrier()
        drain(0, gsem)
        scatter(0)
        gather(2, 2)
        drain(1, gsem)
        scatter(1)
        drain(0, ssem)
        gather(3, 0)
        drain(2, gsem)
        scatter(2)
        drain(1, ssem)
        gather(4, 1)

        def body(i, carry):
            g0 = 3 * i
            drain(0, gsem)
            scatter(0)
            drain(2, ssem)
            gather(g0 + 2, 2)
            drain(1, gsem)
            scatter(1)
            drain(0, ssem)
            gather(g0 + 3, 0)
            drain(2, gsem)
            scatter(2)
            drain(1, ssem)
            gather(g0 + 4, 1)
            return carry

        lax.fori_loop(1, nc // 3 - 1, body, 0)
        drain(0, gsem)
        scatter(0)
        drain(2, ssem)
        gather(nc - 1, 2)
        drain(1, gsem)
        scatter(1)
        drain(0, ssem)
        drain(2, gsem)
        scatter(2)
        drain(1, ssem)
        drain(2, ssem)
        plsc.subcore_barrier()
        pltpu.sync_copy(acc_sh.at[pl.ds(rw0, rows_per_sub)],
                        part_h.at[c, pl.ds(rw0, rows_per_sub)])

    return k(x, packed, zeros)


def _tc_prep(feat, degs, L0):
    n, d = feat.shape

    def body(feat_r, degs_r, l0_r, x0_r, y0_r, ns_r, nd_r):
        deg_o = degs_r[0, :n, 0:1]
        deg_i = degs_r[1, :n, 0:1]
        ns = lax.rsqrt(jnp.maximum(deg_o, 1.0))
        nd = lax.rsqrt(jnp.maximum(deg_i, 1.0))
        f = feat_r[...]
        x0_r[...] = f * ns
        y0_r[...] = jnp.dot(f, l0_r[...], preferred_element_type=jnp.float32)
        ns_r[...] = ns
        nd_r[...] = nd

    return pl.pallas_call(
        body,
        out_shape=[
            jax.ShapeDtypeStruct((n, d), jnp.float32),
            jax.ShapeDtypeStruct((n, L0.shape[1]), jnp.float32),
            jax.ShapeDtypeStruct((n, 1), jnp.float32),
            jax.ShapeDtypeStruct((n, 1), jnp.float32),
        ],
    )(feat, degs, L0)


def _tc_mid(part, nd, ns, W, y_prev, L, b):
    n, d = y_prev.shape
    d_next = L.shape[1]
    has_b = b is not None

    def body(part_r, nd_r, ns_r, w_r, y_r, l_r, *rest):
        if has_b:
            b_r, xn_r, yn_r = rest
        else:
            xn_r, yn_r = rest
        agg = (part_r[0, :n] + part_r[1, :n]) * nd_r[...]
        h = jnp.dot(agg, w_r[...], preferred_element_type=jnp.float32) + y_r[...]
        xn_r[...] = h * ns_r[...]
        yn = jnp.dot(h, l_r[...], preferred_element_type=jnp.float32)
        if has_b:
            yn = yn + b_r[...]
        yn_r[...] = yn

    args = [part, nd, ns, W, y_prev, L]
    if has_b:
        args.append(b.reshape(1, -1))
    return pl.pallas_call(
        body,
        out_shape=[
            jax.ShapeDtypeStruct((n, d), jnp.float32),
            jax.ShapeDtypeStruct((n, d_next), jnp.float32),
        ],
    )(*args)


def _tc_final(part, nd, W, y_prev):
    n, d_out = y_prev.shape

    def body(part_r, nd_r, w_r, y_r, out_r):
        agg = (part_r[0, :n] + part_r[1, :n]) * nd_r[...]
        out_r[...] = (jnp.dot(agg, w_r[...], preferred_element_type=jnp.float32)
                      + y_r[...])

    return pl.pallas_call(
        body,
        out_shape=jax.ShapeDtypeStruct((n, d_out), jnp.float32),
    )(part, nd, W, y_prev)


def kernel(feat, edge_index, W0, W1, W2, b2, L0, L1, L2):
    n, d = feat.shape
    n_pad = ((n + 127) // 128) * 128
    src = edge_index[0]
    dst = edge_index[1]
    e = src.shape[0]
    chunk_sz = _NW * _CS
    nc = -(-e // chunk_sz)
    nc += (-nc) % 3
    e_pad = nc * chunk_sz
    pad_i = jnp.arange(e_pad - e, dtype=jnp.int32)
    src_p = jnp.concatenate([src, pad_i % n])
    dst_p = jnp.concatenate([dst, n + pad_i % (n_pad - n)])
    packed = (src_p | (dst_p << 16)).reshape(_NW, nc, _CS)

    degs = _degrees(src, dst, n_pad)
    x0, y0, ns, nd = _tc_prep(feat, degs, L0)

    p0 = _spmm(x0, packed, n_pad)
    x1, y1 = _tc_mid(p0, nd, ns, W0, y0, L1, None)

    p1 = _spmm(x1, packed, n_pad)
    x2, y2 = _tc_mid(p1, nd, ns, W1, y1, L2, b2)

    p2 = _spmm(x2, packed, n_pad)
    return _tc_final(p2, nd, W2, y2)
